# baseline (device time: 49533 ns/iter reference)
import jax
import jax.numpy as jnp
from jax import lax
from jax.experimental import pallas as pl
from jax.experimental.pallas import tpu as pltpu

NZ = 4
NR = 8
EPS = 1e-6


def kernel(partial, gamma):
    _, m, d = partial.shape
    x = partial.reshape(m, d)
    g = gamma.reshape(1, d)
    m_out = m // NZ
    dsl = d // NR

    def body(x_ref, g_ref, out_ref,
             x_slices, p1_stage, p1_recv, p2_recv,
             load_sems, p1_send_sems, p1_recv_sems, p2_send_sems,
             p2_recv_sems):
        my_x = lax.axis_index("x")
        my_y = lax.axis_index("y")
        my_z = lax.axis_index("z")
        my_q = my_x * 4 + my_y
        col0 = my_q * dsl

        loads = []
        for k in range(NZ):
            zt = lax.rem(my_z + k, NZ)
            cp = pltpu.make_async_copy(
                x_ref.at[pl.ds(zt * m_out, m_out), pl.ds(col0, dsl)],
                x_slices.at[k],
                load_sems.at[k],
            )
            cp.start()
            loads.append(cp)

        barrier = pltpu.get_barrier_semaphore()
        for o in range(1, NZ):
            zt = lax.rem(my_z + o, NZ)
            pl.semaphore_signal(barrier, inc=1, device_id=(my_x, my_y, zt),
                                device_id_type=pl.DeviceIdType.MESH)
        for o in range(1, NR):
            qt = lax.rem(my_q + o, NR)
            pl.semaphore_signal(barrier, inc=1,
                                device_id=(qt // 4, lax.rem(qt, 4), my_z),
                                device_id_type=pl.DeviceIdType.MESH)
        pl.semaphore_wait(barrier, (NZ - 1) + (NR - 1))

        p1_sends = []
        for o in range(1, NZ):
            zt = lax.rem(my_z + o, NZ)
            loads[o].wait()
            p1_stage[o - 1, :, :] = x_slices[o, :, :].astype(jnp.bfloat16)
            send = pltpu.make_async_remote_copy(
                src_ref=p1_stage.at[o - 1],
                dst_ref=p1_recv.at[NZ - o - 1],
                send_sem=p1_send_sems.at[o - 1],
                recv_sem=p1_recv_sems.at[NZ - o - 1],
                device_id=(my_x, my_y, zt),
                device_id_type=pl.DeviceIdType.MESH,
            )
            send.start()
            p1_sends.append(send)

        for s in range(NZ - 1):
            pltpu.make_async_remote_copy(
                src_ref=p1_stage.at[0],
                dst_ref=p1_recv.at[s],
                send_sem=p1_send_sems.at[0],
                recv_sem=p1_recv_sems.at[s],
                device_id=(my_x, my_y, my_z),
                device_id_type=pl.DeviceIdType.MESH,
            ).wait_recv()

        loads[0].wait()
        y_slice = (
            p1_recv[0, :, :].astype(jnp.float32)
            + p1_recv[1, :, :].astype(jnp.float32)
            + p1_recv[2, :, :].astype(jnp.float32)
            + x_slices[0, :, :]
        )
        p2_recv[my_q, :, :] = y_slice.astype(jnp.bfloat16)

        p2_sends = []
        for o in range(1, NR):
            qt = lax.rem(my_q + o, NR)
            send = pltpu.make_async_remote_copy(
                src_ref=p2_recv.at[my_q],
                dst_ref=p2_recv.at[my_q],
                send_sem=p2_send_sems.at[o - 1],
                recv_sem=p2_recv_sems.at[my_q],
                device_id=(qt // 4, lax.rem(qt, 4), my_z),
                device_id_type=pl.DeviceIdType.MESH,
            )
            send.start()
            p2_sends.append(send)

        for o in range(1, NR):
            qs = lax.rem(my_q + o, NR)
            pltpu.make_async_remote_copy(
                src_ref=p2_recv.at[my_q],
                dst_ref=p2_recv.at[qs],
                send_sem=p2_send_sems.at[0],
                recv_sem=p2_recv_sems.at[qs],
                device_id=(my_x, my_y, my_z),
                device_id_type=pl.DeviceIdType.MESH,
            ).wait_recv()

        ssq = jnp.zeros((m_out, 1), jnp.float32)
        for j in range(NR):
            yj = p2_recv[j, :, :].astype(jnp.float32)
            ssq = ssq + jnp.sum(yj * yj, axis=1, keepdims=True)
        inv = lax.rsqrt(ssq / d + EPS)
        for j in range(NR):
            out_ref[:, j * dsl:(j + 1) * dsl] = (
                p2_recv[j, :, :].astype(jnp.float32)
                * inv * g_ref[:, j * dsl:(j + 1) * dsl]
            )

        for send in p1_sends + p2_sends:
            send.wait_send()

    return pl.pallas_call(
        body,
        out_shape=jax.ShapeDtypeStruct((m_out, d), jnp.float32),
        in_specs=[
            pl.BlockSpec(memory_space=pltpu.MemorySpace.HBM),
            pl.BlockSpec(memory_space=pltpu.VMEM),
        ],
        out_specs=pl.BlockSpec(memory_space=pltpu.VMEM),
        scratch_shapes=[
            pltpu.VMEM((NZ, m_out, dsl), jnp.float32),
            pltpu.VMEM((NZ - 1, m_out, dsl), jnp.bfloat16),
            pltpu.VMEM((NZ - 1, m_out, dsl), jnp.bfloat16),
            pltpu.VMEM((NR, m_out, dsl), jnp.bfloat16),
            pltpu.SemaphoreType.DMA((NZ,)),
            pltpu.SemaphoreType.DMA((NZ - 1,)),
            pltpu.SemaphoreType.DMA((NZ - 1,)),
            pltpu.SemaphoreType.DMA((NR - 1,)),
            pltpu.SemaphoreType.DMA((NR,)),
        ],
        compiler_params=pltpu.CompilerParams(collective_id=0),
    )(x, g)


# device time: 44569 ns/iter; 1.1114x vs baseline; 1.1114x over previous
import jax
import jax.numpy as jnp
from jax import lax
from jax.experimental import pallas as pl
from jax.experimental.pallas import tpu as pltpu

NZ = 4
NY = 4
NX = 2
NR = NX * NY
EPS = 1e-6


def kernel(partial, gamma):
    _, m, d = partial.shape
    x = partial.reshape(m, d)
    g = gamma.reshape(1, d)
    m_out = m // NZ
    dsl = d // NR

    def body(x_ref, g_ref, out_ref,
             x_slices, p1_stage, p1_recv, p2_recv,
             load_sems, p1_send_sems, p1_recv_sems,
             p2s_own, p2s_x, p2s_f, p2_recv_sems):
        my_x = lax.axis_index("x")
        my_y = lax.axis_index("y")
        my_z = lax.axis_index("z")
        my_q = my_x * NY + my_y
        op_x = 1 - my_x
        col0 = my_q * dsl

        sends = []

        def remote(src, dst, ssem, rsem, dev):
            return pltpu.make_async_remote_copy(
                src_ref=src, dst_ref=dst, send_sem=ssem, recv_sem=rsem,
                device_id=dev, device_id_type=pl.DeviceIdType.MESH,
            )

        def start_if(cond, rdma):
            if cond is None:
                rdma.start()
            else:
                @pl.when(cond)
                def _():
                    rdma.start()
            sends.append((cond, rdma))

        def wait_slot(slot):
            remote(p2_recv.at[slot], p2_recv.at[slot],
                   p2s_own.at[0], p2_recv_sems.at[slot],
                   (my_x, my_y, my_z)).wait_recv()

        loads = []
        for k in range(NZ):
            zt = lax.rem(my_z + k, NZ)
            cp = pltpu.make_async_copy(
                x_ref.at[pl.ds(zt * m_out, m_out), pl.ds(col0, dsl)],
                x_slices.at[k],
                load_sems.at[k],
            )
            cp.start()
            loads.append(cp)

        with jax.named_scope("phase0_barrier"):
            barrier = pltpu.get_barrier_semaphore()
            for o in range(1, NZ):
                zt = lax.rem(my_z + o, NZ)
                pl.semaphore_signal(barrier, inc=1, device_id=(my_x, my_y, zt),
                                    device_id_type=pl.DeviceIdType.MESH)
            pl.semaphore_signal(barrier, inc=1, device_id=(op_x, my_y, my_z),
                                device_id_type=pl.DeviceIdType.MESH)
            for dy in (-2, -1, 1, 2):
                ok = (my_y + dy >= 0) & (my_y + dy < NY)
                yt = jnp.where(ok, jnp.clip(my_y + dy, 0, NY - 1), my_y)
                pl.semaphore_signal(barrier, inc=1, device_id=(my_x, yt, my_z),
                                    device_id_type=pl.DeviceIdType.MESH)
            pl.semaphore_wait(barrier, NZ - 1 + 1 + 4)

        with jax.named_scope("phase1_send"):
            for o in range(1, NZ):
                zt = lax.rem(my_z + o, NZ)
                loads[o].wait()
                p1_stage[o - 1, :, :] = x_slices[o, :, :].astype(jnp.bfloat16)
                rdma = remote(p1_stage.at[o - 1], p1_recv.at[NZ - o - 1],
                              p1_send_sems.at[o - 1],
                              p1_recv_sems.at[NZ - o - 1],
                              (my_x, my_y, zt))
                rdma.start()
                sends.append((None, rdma))

        with jax.named_scope("phase1_waitrecv"):
            for s in range(NZ - 1):
                remote(p1_stage.at[0], p1_recv.at[s], p1_send_sems.at[0],
                       p1_recv_sems.at[s], (my_x, my_y, my_z)).wait_recv()

        with jax.named_scope("phase1_reduce"):
            loads[0].wait()
            y_slice = (
                p1_recv[0, :, :].astype(jnp.float32)
                + p1_recv[1, :, :].astype(jnp.float32)
                + p1_recv[2, :, :].astype(jnp.float32)
                + x_slices[0, :, :]
            )
            p2_recv[my_q, :, :] = y_slice.astype(jnp.bfloat16)

        with jax.named_scope("phase2_own_sends"):
            own = p2_recv.at[my_q]
            start_if(None, remote(own, own, p2s_own.at[4],
                                  p2_recv_sems.at[my_q],
                                  (op_x, my_y, my_z)))
            for i, dy in enumerate((1, 2, -1, -2)):
                ok = (my_y + dy >= 0) & (my_y + dy < NY)
                yt = jnp.where(ok, jnp.clip(my_y + dy, 0, NY - 1), my_y)
                start_if(ok, remote(own, own, p2s_own.at[i],
                                    p2_recv_sems.at[my_q],
                                    (my_x, yt, my_z)))

        with jax.named_scope("phase2_streams"):
            steps = [
                ((my_y >= 1), my_y - 1, 0, None),
                ((my_y <= NY - 2), my_y + 1, 1, None),
                ((my_y >= 2), my_y - 2, 2, "right"),
                ((my_y <= NY - 3), my_y + 2, 3, "left"),
                ((my_y == NY - 1), my_y - 3, 4, None),
                ((my_y == 0), my_y + 3, 5, None),
            ]
            for cond, j, xsem, yfwd in steps:
                jc = jnp.clip(j, 0, NY - 1)
                slot = my_x * NY + jc

                @pl.when(cond)
                def _(slot=slot):
                    wait_slot(slot)

                start_if(cond, remote(p2_recv.at[slot], p2_recv.at[slot],
                                      p2s_x.at[xsem], p2_recv_sems.at[slot],
                                      (op_x, my_y, my_z)))
                if yfwd == "right":
                    c = cond & (my_y == 2)
                    start_if(c, remote(p2_recv.at[slot], p2_recv.at[slot],
                                       p2s_f.at[0], p2_recv_sems.at[slot],
                                       (my_x, jnp.int32(NY - 1), my_z)))
                elif yfwd == "left":
                    c = cond & (my_y == 1)
                    start_if(c, remote(p2_recv.at[slot], p2_recv.at[slot],
                                       p2s_f.at[1], p2_recv_sems.at[slot],
                                       (my_x, jnp.int32(0), my_z)))

        with jax.named_scope("phase2_waitrecv_x"):
            for jp in range(NY):
                wait_slot(op_x * NY + jp)

        with jax.named_scope("phase3_norm"):
            ssq = jnp.zeros((m_out, 1), jnp.float32)
            for j in range(NR):
                yj = p2_recv[j, :, :].astype(jnp.float32)
                ssq = ssq + jnp.sum(yj * yj, axis=1, keepdims=True)
            inv = lax.rsqrt(ssq / d + EPS)
            for j in range(NR):
                out_ref[:, j * dsl:(j + 1) * dsl] = (
                    p2_recv[j, :, :].astype(jnp.float32)
                    * inv * g_ref[:, j * dsl:(j + 1) * dsl]
                )

        with jax.named_scope("drain_sends"):
            for cond, rdma in sends:
                if cond is None:
                    rdma.wait_send()
                else:
                    @pl.when(cond)
                    def _(rdma=rdma):
                        rdma.wait_send()

    return pl.pallas_call(
        body,
        out_shape=jax.ShapeDtypeStruct((m_out, d), jnp.float32),
        in_specs=[
            pl.BlockSpec(memory_space=pltpu.MemorySpace.HBM),
            pl.BlockSpec(memory_space=pltpu.VMEM),
        ],
        out_specs=pl.BlockSpec(memory_space=pltpu.VMEM),
        scratch_shapes=[
            pltpu.VMEM((NZ, m_out, dsl), jnp.float32),
            pltpu.VMEM((NZ - 1, m_out, dsl), jnp.bfloat16),
            pltpu.VMEM((NZ - 1, m_out, dsl), jnp.bfloat16),
            pltpu.VMEM((NR, m_out, dsl), jnp.bfloat16),
            pltpu.SemaphoreType.DMA((NZ,)),
            pltpu.SemaphoreType.DMA((NZ - 1,)),
            pltpu.SemaphoreType.DMA((NZ - 1,)),
            pltpu.SemaphoreType.DMA((5,)),
            pltpu.SemaphoreType.DMA((6,)),
            pltpu.SemaphoreType.DMA((2,)),
            pltpu.SemaphoreType.DMA((NR,)),
        ],
        compiler_params=pltpu.CompilerParams(collective_id=0),
    )(x, g)
